# baseline (speedup 1.0000x reference)
import functools

import jax
import jax.numpy as jnp
from jax import lax
from jax.experimental import pallas as pl
from jax.experimental.pallas import tpu as pltpu
from jax.experimental.pallas import tpu_sc as plsc

_V = 1000000
_D = 64
_B = 4096
_L = 20
_NNEG = 5

_NTEC = 16
_BPT = _B // _NTEC
_DPC = _D // 2
_NSC = _B * (1 + _NNEG)


def _sc_body(in_t, out_c, ctx_ids, in_hi, in_lo, neg_hi, neg_lo, zidx_in, out,
             sh0, ctx_ids_v, in_hi_v, in_lo_v, neg_hi_v, neg_lo_v, zidx,
             ctxval, posc, negc, zc_v, inv_v, ce_v, pacc, nacc, r0v,
             sem_s0, sem_g, sem_h):
    c = lax.axis_index("c")
    s = lax.axis_index("s")
    lanes = lax.iota(jnp.int32, 16)
    zerosf = jnp.zeros((16,), jnp.float32)

    pltpu.sync_copy(ctx_ids.at[pl.ds(s * (_BPT * _L), _BPT * _L)], ctx_ids_v)
    pltpu.sync_copy(in_hi.at[pl.ds(s * _BPT, _BPT)], in_hi_v)
    pltpu.sync_copy(in_lo.at[pl.ds(s * _BPT, _BPT)], in_lo_v)
    pltpu.sync_copy(neg_hi.at[pl.ds(s * (_BPT * _NNEG), _BPT * _NNEG)],
                    neg_hi_v)
    pltpu.sync_copy(neg_lo.at[pl.ds(s * (_BPT * _NNEG), _BPT * _NNEG)],
                    neg_lo_v)
    pltpu.sync_copy(zidx_in, zidx)

    for blk in range(_BPT // 16):
        boff = blk * 16
        zc = zerosf
        for l in range(_L):
            idv = ctx_ids_v[pl.ds(l * _BPT + boff, 16)]
            zc = zc + jnp.where(idv == 0, 1.0, 0.0)
        inv = jnp.where(zc >= jnp.float32(_L), 0.0,
                        1.0 / ((jnp.float32(_L) - zc) + 1e-9))
        zc_v[pl.ds(boff, 16)] = zc
        inv_v[pl.ds(boff, 16)] = inv
        pacc[pl.ds(boff, 16)] = zerosf
        for n in range(_NNEG):
            nacc[pl.ds(n * _BPT + boff, 16)] = zerosf

    @pl.when(s == 0)
    def _():
        pltpu.async_copy(in_t.at[c * _DPC], sh0, sem_s0)

    def dstep(j, carry):
        dd = c * _DPC + j

        hb = []
        for k in range(_BPT // 128):
            hb.append(pltpu.async_copy(
                out_c.at[dd].at[in_hi_v.at[pl.ds(k * 128, 128)]],
                posc.at[pl.ds(k * 128, 128)], sem_h))
        for k in range(_BPT * _NNEG // 128):
            hb.append(pltpu.async_copy(
                out_c.at[dd].at[neg_hi_v.at[pl.ds(k * 128, 128)]],
                negc.at[pl.ds(k * 128, 128)], sem_h))

        @pl.when(s == 0)
        def _():
            pltpu.make_async_copy(in_t.at[dd], sh0, sem_s0).wait()

        plsc.subcore_barrier()

        hs = [pltpu.async_copy(sh0.at[zidx], r0v, sem_g)]
        for k in range(_BPT * _L // 128):
            hs.append(pltpu.async_copy(
                sh0.at[ctx_ids_v.at[pl.ds(k * 128, 128)]],
                ctxval.at[pl.ds(k * 128, 128)], sem_g))
        for h in hs:
            h.wait()

        r0 = r0v[...]
        for blk in range(_BPT // 16):
            boff = blk * 16
            acc = ctxval[pl.ds(boff, 16)]
            for l in range(1, _L):
                acc = acc + ctxval[pl.ds(l * _BPT + boff, 16)]
            zc = zc_v[pl.ds(boff, 16)]
            inv = inv_v[pl.ds(boff, 16)]
            ce_v[pl.ds(boff, 16)] = (acc - zc * r0) * inv

        plsc.subcore_barrier()

        @pl.when((s == 0) & (j < _DPC - 1))
        def _():
            pltpu.async_copy(in_t.at[dd + 1], sh0, sem_s0)

        for h in hb:
            h.wait()

        for blk in range(_BPT // 16):
            boff = blk * 16
            ce = ce_v[pl.ds(boff, 16)]
            plo = in_lo_v[pl.ds(boff, 16)]
            pv = plsc.load_gather(posc, [boff + lanes, plo])
            pacc[pl.ds(boff, 16)] = pacc[pl.ds(boff, 16)] + ce * pv
            for n in range(_NNEG):
                noff = n * _BPT + boff
                nlo = neg_lo_v[pl.ds(noff, 16)]
                nv = plsc.load_gather(negc, [noff + lanes, nlo])
                nacc[pl.ds(noff, 16)] = nacc[pl.ds(noff, 16)] - ce * nv

        plsc.subcore_barrier()
        return carry

    lax.fori_loop(0, _DPC, dstep, 0)

    base = c * _NSC
    pltpu.sync_copy(pacc, out.at[pl.ds(base + s * _BPT, _BPT)])
    pltpu.sync_copy(nacc, out.at[pl.ds(base + _B + s * (_BPT * _NNEG),
                                       _BPT * _NNEG)])


_sc_scores = functools.partial(
    pl.kernel,
    out_type=jax.ShapeDtypeStruct((2 * _NSC,), jnp.float32),
    mesh=plsc.VectorSubcoreMesh(core_axis_name="c", subcore_axis_name="s"),
    scratch_types=[
        pltpu.VMEM_SHARED((_V,), jnp.float32),
        pltpu.VMEM((_BPT * _L,), jnp.int32),
        pltpu.VMEM((_BPT,), jnp.int32),
        pltpu.VMEM((_BPT,), jnp.int32),
        pltpu.VMEM((_BPT * _NNEG,), jnp.int32),
        pltpu.VMEM((_BPT * _NNEG,), jnp.int32),
        pltpu.VMEM((16,), jnp.int32),
        pltpu.VMEM((_BPT * _L,), jnp.float32),
        pltpu.VMEM((_BPT, 16), jnp.float32),
        pltpu.VMEM((_BPT * _NNEG, 16), jnp.float32),
        pltpu.VMEM((_BPT,), jnp.float32),
        pltpu.VMEM((_BPT,), jnp.float32),
        pltpu.VMEM((_BPT,), jnp.float32),
        pltpu.VMEM((_BPT,), jnp.float32),
        pltpu.VMEM((_BPT * _NNEG,), jnp.float32),
        pltpu.VMEM((16,), jnp.float32),
        pltpu.SemaphoreType.DMA,
        pltpu.SemaphoreType.DMA,
        pltpu.SemaphoreType.DMA,
    ],
    compiler_params=pltpu.CompilerParams(
        needs_layout_passes=False, use_tc_tiling_on_sc=False),
)(_sc_body)


def _tc_loss_body(x_ref, o_ref):
    x = x_ref[...]
    half = _NSC // 128
    t = x[:half, :] + x[half:, :]
    ls = jnp.minimum(t, 0.0) - jnp.log(1.0 + jnp.exp(-jnp.abs(t)))
    o_ref[0, 0] = -(jnp.sum(ls) / jnp.float32(_B))


_tc_loss = pl.pallas_call(
    _tc_loss_body,
    out_shape=jax.ShapeDtypeStruct((1, 1), jnp.float32),
    out_specs=pl.BlockSpec(memory_space=pltpu.SMEM),
)


def kernel(in_emb, out_emb, input_ids, context_ids, negative_ids):
    ctx_r = (context_ids.astype(jnp.int32)
             .reshape(_NTEC, _BPT, _L).transpose(0, 2, 1).reshape(-1))
    neg_r = (negative_ids.astype(jnp.int32)
             .reshape(_NTEC, _BPT, _NNEG).transpose(0, 2, 1).reshape(-1))
    in_flat = input_ids.astype(jnp.int32)
    zidx = jnp.zeros((16,), jnp.int32)
    out_c = out_emb.T.reshape(_D, _V // 16, 16)
    scores = _sc_scores(in_emb.T, out_c, ctx_r,
                        in_flat >> 4, in_flat & 15,
                        neg_r >> 4, neg_r & 15, zidx)
    loss = _tc_loss(scores.reshape(2 * _NSC // 128, 128))
    return loss[0, 0]
